# Initial kernel scaffold; baseline (speedup 1.0000x reference)
import jax
import jax.numpy as jnp
from jax.experimental import pallas as pl


def kernel(x, table):
    raise NotImplementedError("write your pallas kernel here")


# speedup vs baseline: 1.2557x; 1.2557x over previous
import functools

import jax
import jax.numpy as jnp
from jax import lax
from jax.experimental import pallas as pl
from jax.experimental.pallas import tpu as pltpu
from jax.experimental.pallas import tpu_sc as plsc

NC = 2
NS = 16
NW = NC * NS

K = 16


@functools.lru_cache(maxsize=None)
def _make_emb(B: int, D: int):
    bpw = B // NW
    nchunk = bpw // K
    mesh = plsc.VectorSubcoreMesh(core_axis_name="c", subcore_axis_name="s")

    @functools.partial(
        pl.kernel,
        mesh=mesh,
        out_type=jax.ShapeDtypeStruct((B, D), jnp.float32),
        scratch_types=[
            pltpu.VMEM((nchunk, K), jnp.int32),
            pltpu.VMEM((K, D), jnp.float32),
            pltpu.SemaphoreType.DMA,
        ],
    )
    def emb(table_hbm, idx_hbm, out_hbm, idx_v, buf, gsem):
        wid = lax.axis_index("s") * NC + lax.axis_index("c")
        row_base = wid * bpw
        pltpu.sync_copy(idx_hbm.at[pl.ds(wid * nchunk, nchunk)], idx_v)

        def body(c, carry):
            pltpu.async_copy(table_hbm.at[idx_v.at[c]], buf, gsem).wait()
            pltpu.sync_copy(buf, out_hbm.at[pl.ds(row_base + c * K, K)])
            return carry

        lax.fori_loop(0, nchunk, body, 0)

    return emb


def kernel(x, table):
    B = x.size
    D = table.shape[1]
    idx2d = x.reshape(B // K, K).astype(jnp.int32)
    out = _make_emb(B, D)(table, idx2d)
    return out.reshape(x.shape + (D,))


# speedup vs baseline: 1.3111x; 1.0441x over previous
import functools

import jax
import jax.numpy as jnp
from jax import lax
from jax.experimental import pallas as pl
from jax.experimental.pallas import tpu as pltpu
from jax.experimental.pallas import tpu_sc as plsc

NC = 2
NS = 16
NW = NC * NS

K = 16
NBUF = 2


@functools.lru_cache(maxsize=None)
def _make_emb(B: int, D: int):
    bpw = B // NW
    nchunk = bpw // K
    assert nchunk >= 2 * NBUF
    mesh = plsc.VectorSubcoreMesh(core_axis_name="c", subcore_axis_name="s")

    @functools.partial(
        pl.kernel,
        mesh=mesh,
        out_type=jax.ShapeDtypeStruct((B, D), jnp.float32),
        scratch_types=[
            pltpu.VMEM((nchunk, K), jnp.int32),
            pltpu.VMEM((K, D), jnp.float32),
            pltpu.VMEM((K, D), jnp.float32),
            pltpu.SemaphoreType.DMA,
            pltpu.SemaphoreType.DMA,
            pltpu.SemaphoreType.DMA,
            pltpu.SemaphoreType.DMA,
        ],
    )
    def emb(table_hbm, idx_hbm, out_hbm, idx_v, buf0, buf1, g0, g1, s0, s1):
        wid = lax.axis_index("s") * NC + lax.axis_index("c")
        row_base = wid * bpw
        bufs = (buf0, buf1)
        gsem = (g0, g1)
        ssem = (s0, s1)
        pltpu.sync_copy(idx_hbm.at[pl.ds(wid * nchunk, nchunk)], idx_v)

        def gather_start(c, b):
            pltpu.async_copy(table_hbm.at[idx_v.at[c]], bufs[b], gsem[b])

        def gather_wait(c, b):
            pltpu.make_async_copy(
                table_hbm.at[idx_v.at[c]], bufs[b], gsem[b]).wait()

        def out_slice(c):
            return out_hbm.at[pl.ds(row_base + c * K, K)]

        def scatter_start(c, b):
            pltpu.async_copy(bufs[b], out_slice(c), ssem[b])

        def scatter_wait(c, b):
            pltpu.make_async_copy(bufs[b], out_slice(c), ssem[b]).wait()

        for b in range(NBUF):
            gather_start(b, b)

        def outer(g, carry):
            for b in range(NBUF):
                c = NBUF * g + b
                gather_wait(c, b)
                scatter_start(c, b)
                scatter_wait(c, b)
                gather_start(c + NBUF, b)
            return carry

        lax.fori_loop(0, nchunk // NBUF - 1, outer, 0)

        for b in range(NBUF):
            c = nchunk - NBUF + b
            gather_wait(c, b)
            scatter_start(c, b)
        for b in range(NBUF):
            scatter_wait(nchunk - NBUF + b, b)

    return emb


def kernel(x, table):
    B = x.size
    D = table.shape[1]
    idx2d = x.reshape(B // K, K).astype(jnp.int32)
    out = _make_emb(B, D)(table, idx2d)
    return out.reshape(x.shape + (D,))


# speedup vs baseline: 1.6903x; 1.2892x over previous
import functools

import jax
import jax.numpy as jnp
from jax import lax
from jax.experimental import pallas as pl
from jax.experimental.pallas import tpu as pltpu
from jax.experimental.pallas import tpu_sc as plsc

NC = 2
NS = 16
NW = NC * NS

SA = 16
ST = 4
PSTRIDE = 64


@functools.lru_cache(maxsize=None)
def _make_emb(N: int, S: int, D: int):
    npw = N // NW
    assert npw % 2 == 0
    npairs = npw // 2
    mesh = plsc.VectorSubcoreMesh(core_axis_name="c", subcore_axis_name="s")

    @functools.partial(
        pl.kernel,
        mesh=mesh,
        out_type=(
            jax.ShapeDtypeStruct((N, S, D), jnp.float32),
            jax.ShapeDtypeStruct((N * ST, D), jnp.float32),
        ),
        scratch_types=[
            pltpu.VMEM((npairs * PSTRIDE,), jnp.int32),
            pltpu.VMEM((SA, D), jnp.float32),
            pltpu.VMEM((2 * ST, D), jnp.float32),
            pltpu.SemaphoreType.DMA,
            pltpu.SemaphoreType.DMA,
        ],
    )
    def emb(table_hbm, idx_hbm, out_hbm, tails_hbm, idx_v, buf, tbuf, g0, g1):
        wid = lax.axis_index("s") * NC + lax.axis_index("c")
        base = wid * npw
        pltpu.sync_copy(
            idx_hbm.at[pl.ds(wid * npairs * PSTRIDE, npairs * PSTRIDE)],
            idx_v)

        def body(p, carry):
            q = p * PSTRIDE
            pltpu.async_copy(
                table_hbm.at[idx_v.at[pl.ds(q + 2 * SA, 2 * ST)]], tbuf, g1)
            for h in range(2):
                c = 2 * p + h
                pltpu.async_copy(
                    table_hbm.at[idx_v.at[pl.ds(q + h * SA, SA)]], buf, g0)
                pltpu.make_async_copy(
                    table_hbm.at[idx_v.at[pl.ds(q + h * SA, SA)]], buf,
                    g0).wait()
                pltpu.sync_copy(buf, out_hbm.at[base + c, pl.ds(0, SA)])
            pltpu.make_async_copy(
                table_hbm.at[idx_v.at[pl.ds(q + 2 * SA, 2 * ST)]], tbuf,
                g1).wait()
            pltpu.sync_copy(
                tbuf, tails_hbm.at[pl.ds((base + 2 * p) * ST, 2 * ST)])
            return carry

        lax.fori_loop(0, npairs, body, 0)

    return emb


def _tc_fix_body(tails_ref, big_any, out_ref):
    for k in range(out_ref.shape[0]):
        out_ref[k, pl.ds(0, ST)] = tails_ref[pl.ds(k * ST, ST)]


@functools.lru_cache(maxsize=None)
def _make_fix(N: int, S: int, D: int):
    BI = 16
    return pl.pallas_call(
        _tc_fix_body,
        grid=(N // BI,),
        in_specs=[
            pl.BlockSpec((BI * ST, D), lambda i: (i, 0)),
            pl.BlockSpec(memory_space=pl.ANY),
        ],
        out_specs=pl.BlockSpec((BI, 8, D), lambda i: (i, 2, 0)),
        out_shape=jax.ShapeDtypeStruct((N, S, D), jnp.float32),
        input_output_aliases={1: 0},
    )


def kernel(x, table):
    N, S = x.shape
    D = table.shape[1]
    npw = N // NW
    x4 = x.astype(jnp.int32).reshape(NW, npw // 2, 2, S)
    main = x4[..., :SA].reshape(NW, npw // 2, 2 * SA)
    tails_idx = x4[..., SA:].reshape(NW, npw // 2, 2 * ST)
    pad = jnp.zeros((NW, npw // 2, PSTRIDE - 2 * SA - 2 * ST), jnp.int32)
    xp = jnp.concatenate([main, tails_idx, pad], axis=-1).reshape(-1)
    big, tails = _make_emb(N, S, D)(table, xp)
    return _make_fix(N, S, D)(tails, big)


# speedup vs baseline: 1.7851x; 1.0561x over previous
import functools

import jax
import jax.numpy as jnp
from jax import lax
from jax.experimental import pallas as pl
from jax.experimental.pallas import tpu as pltpu
from jax.experimental.pallas import tpu_sc as plsc

NC = 2
NS = 16
NW = NC * NS

SA = 16
ST = 4
PSTRIDE = 64


@functools.lru_cache(maxsize=None)
def _make_emb(N: int, S: int, D: int):
    npw = N // NW
    assert npw % 2 == 0
    npairs = npw // 2
    mesh = plsc.VectorSubcoreMesh(core_axis_name="c", subcore_axis_name="s")

    @functools.partial(
        pl.kernel,
        mesh=mesh,
        out_type=(
            jax.ShapeDtypeStruct((N, S, D), jnp.float32),
            jax.ShapeDtypeStruct((N * ST, D), jnp.float32),
        ),
        scratch_types=[
            pltpu.VMEM((npairs * PSTRIDE,), jnp.int32),
            pltpu.VMEM((SA, D), jnp.float32),
            pltpu.VMEM((SA, D), jnp.float32),
            pltpu.VMEM((2 * ST, D), jnp.float32),
            pltpu.VMEM((2 * ST, D), jnp.float32),
            pltpu.SemaphoreType.DMA,
            pltpu.SemaphoreType.DMA,
            pltpu.SemaphoreType.DMA,
            pltpu.SemaphoreType.DMA,
            pltpu.SemaphoreType.DMA,
            pltpu.SemaphoreType.DMA,
            pltpu.SemaphoreType.DMA,
            pltpu.SemaphoreType.DMA,
        ],
    )
    def emb(table_hbm, idx_hbm, out_hbm, tails_hbm, idx_v,
            mb0, mb1, tb0, tb1, g0, g1, s0, s1, tg0, tg1, w0, w1):
        wid = lax.axis_index("s") * NC + lax.axis_index("c")
        base = wid * npw
        mbufs, gsem, ssem = (mb0, mb1), (g0, g1), (s0, s1)
        tbufs, tgsem, wsem = (tb0, tb1), (tg0, tg1), (w0, w1)
        pltpu.sync_copy(
            idx_hbm.at[pl.ds(wid * npairs * PSTRIDE, npairs * PSTRIDE)],
            idx_v)

        def midx(c):
            return idx_v.at[pl.ds((c // 2) * PSTRIDE + (c % 2) * SA, SA)]

        def tidx(p):
            return idx_v.at[pl.ds(p * PSTRIDE + 2 * SA, 2 * ST)]

        def mg_start(c, b):
            pltpu.async_copy(table_hbm.at[midx(c)], mbufs[b], gsem[b])

        def mg_wait(c, b):
            pltpu.make_async_copy(
                table_hbm.at[midx(c)], mbufs[b], gsem[b]).wait()

        def ms_slice(c):
            return out_hbm.at[base + c, pl.ds(0, SA)]

        def tg_start(p, t):
            pltpu.async_copy(table_hbm.at[tidx(p)], tbufs[t], tgsem[t])

        def tg_wait(p, t):
            pltpu.make_async_copy(
                table_hbm.at[tidx(p)], tbufs[t], tgsem[t]).wait()

        def ts_slice(p):
            return tails_hbm.at[pl.ds((base + 2 * p) * ST, 2 * ST)]

        mg_start(0, 0)
        mg_start(1, 1)
        tg_start(0, 0)
        tg_start(1, 1)

        def body(p, carry):
            for h in range(2):
                c = 2 * p + h
                b = h
                mg_wait(c, b)
                pltpu.async_copy(mbufs[b], ms_slice(c), ssem[b])
                pltpu.make_async_copy(mbufs[b], ms_slice(c), ssem[b]).wait()

                @pl.when(p < npairs - 1)
                def _():
                    mg_start(c + 2, b)

            t = lax.rem(p, 2)
            for tt in range(2):

                @pl.when(t == tt)
                def _():
                    tg_wait(p, tt)
                    pltpu.async_copy(tbufs[tt], ts_slice(p), wsem[tt])
                    pltpu.make_async_copy(
                        tbufs[tt], ts_slice(p), wsem[tt]).wait()

                    @pl.when(p < npairs - 2)
                    def _():
                        tg_start(p + 2, tt)

            return carry

        lax.fori_loop(0, npairs, body, 0)

    return emb


def _tc_fix_body(tails_ref, big_any, out_ref):
    for k in range(out_ref.shape[0]):
        out_ref[k, pl.ds(0, ST)] = tails_ref[pl.ds(k * ST, ST)]


@functools.lru_cache(maxsize=None)
def _make_fix(N: int, S: int, D: int):
    BI = 16
    return pl.pallas_call(
        _tc_fix_body,
        grid=(N // BI,),
        in_specs=[
            pl.BlockSpec((BI * ST, D), lambda i: (i, 0)),
            pl.BlockSpec(memory_space=pl.ANY),
        ],
        out_specs=pl.BlockSpec((BI, 8, D), lambda i: (i, 2, 0)),
        out_shape=jax.ShapeDtypeStruct((N, S, D), jnp.float32),
        input_output_aliases={1: 0},
    )


def kernel(x, table):
    N, S = x.shape
    D = table.shape[1]
    npw = N // NW
    x4 = x.astype(jnp.int32).reshape(NW, npw // 2, 2, S)
    main = x4[..., :SA].reshape(NW, npw // 2, 2 * SA)
    tails_idx = x4[..., SA:].reshape(NW, npw // 2, 2 * ST)
    pad = jnp.zeros((NW, npw // 2, PSTRIDE - 2 * SA - 2 * ST), jnp.int32)
    xp = jnp.concatenate([main, tails_idx, pad], axis=-1).reshape(-1)
    big, tails = _make_emb(N, S, D)(table, xp)
    return _make_fix(N, S, D)(tails, big)
